# baseline (speedup 1.0000x reference)
import functools

import jax
import jax.numpy as jnp
from jax.experimental import pallas as pl
from jax.experimental.pallas import tpu as pltpu

_INV_SQRT2 = 0.7071067811865476
_MiB = 1024 * 1024


def _gelu(v):
    return 0.5 * v * (1.0 + jax.lax.erf(v * _INV_SQRT2))


def _se_tile_body(x_ref, w1_ref, w2_ref, o_ref, *, inv_s, s, mask_lanes):
    x = x_ref[...]
    if mask_lanes:
        lane = jax.lax.broadcasted_iota(jnp.int32, x.shape, 2)
        x = jnp.where(lane < s, x, 0.0)
    pooled = jnp.sum(x, axis=2) * inv_s

    h = jax.lax.dot_general(pooled, w1_ref[...], (((1,), (1,)), ((), ())),
                            preferred_element_type=jnp.float32)
    h = _gelu(h)
    z = jax.lax.dot_general(h, w2_ref[...], (((1,), (1,)), ((), ())),
                            preferred_element_type=jnp.float32)
    gate = jax.nn.sigmoid(z)

    o_ref[...] = x_ref[...] * gate[:, :, None]


def kernel(x, fc1_w, fc2_w):
    b, c, d, h, w = x.shape
    s = d * h * w
    cr = fc1_w.shape[0]
    x2 = x.reshape(b, c, s)

    s_pad = ((s + 127) // 128) * 128
    tile_bytes = c * s_pad * 4
    budget = 44 * _MiB
    max_bt = max(1, budget // (4 * tile_bytes))
    if b >= 2:
        max_bt = min(max_bt, b // 2)
    bt = 1
    for cand in range(1, b + 1):
        if b % cand == 0 and cand <= max_bt:
            bt = cand
    grid = b // bt

    body = functools.partial(_se_tile_body, inv_s=1.0 / float(s), s=s,
                             mask_lanes=(s % 128 != 0))
    vmem_limit = min(56 * _MiB, 4 * bt * tile_bytes + 4 * c * cr * 4 + 2 * _MiB)

    out2 = pl.pallas_call(
        body,
        out_shape=jax.ShapeDtypeStruct((b, c, s), x.dtype),
        grid=(grid,),
        in_specs=[
            pl.BlockSpec((bt, c, s), lambda i: (i, 0, 0)),
            pl.BlockSpec((cr, c), lambda i: (0, 0)),
            pl.BlockSpec((c, cr), lambda i: (0, 0)),
        ],
        out_specs=pl.BlockSpec((bt, c, s), lambda i: (i, 0, 0)),
        compiler_params=pltpu.CompilerParams(
            dimension_semantics=("parallel",),
            vmem_limit_bytes=int(max(32 * _MiB, vmem_limit)),
        ),
        cost_estimate=pl.CostEstimate(
            flops=2 * b * c * s + 4 * b * c * cr,
            transcendentals=b * (c + cr),
            bytes_accessed=2 * b * c * s * 4 + 2 * c * cr * 4,
        ),
    )(x2, fc1_w, fc2_w)
    return out2.reshape(b, c, d, h, w)
